# baseline (speedup 1.0000x reference)
import jax
import jax.numpy as jnp
from jax import lax
from jax.experimental import pallas as pl
from jax.experimental.pallas import tpu as pltpu
from jax.experimental.pallas import tpu_sc as plsc

SEQ_LEN = 200
EMBED_DIM = 32
BATCH = 4096

NUM_CORES = 2
NUM_SUBCORES = 16
NUM_WORKERS = NUM_CORES * NUM_SUBCORES
BPW = BATCH // NUM_WORKERS

L_BLK = 2
CHUNK = L_BLK * BPW
NUM_CHUNKS = SEQ_LEN // L_BLK
NBUF = 4
NUM_STEPS = NUM_CHUNKS // NBUF
NVREG = CHUNK // 16
OPITCH = BPW + 1


def _body(idxT_hbm, tbl_hbm, pos_hbm, out_hbm,
          idx_v, pos_v, g0, g1, g2, g3, o0, o1, o2, o3,
          sup0, sup1, sup2, sup3,
          psem, gsem0, gsem1, gsem2, gsem3, ssem0, ssem1, ssem2, ssem3):
    G = (g0, g1, g2, g3)
    O = (o0, o1, o2, o3)
    SUP = (sup0, sup1, sup2, sup3)
    GSEM = (gsem0, gsem1, gsem2, gsem3)
    SSEM = (ssem0, ssem1, ssem2, ssem3)

    wid = lax.axis_index("s") * NUM_CORES + lax.axis_index("c")
    bbase = wid * BPW

    pltpu.sync_copy(pos_hbm, pos_v)
    pltpu.async_copy(idxT_hbm.at[:, pl.ds(bbase, BPW)], idx_v, psem).wait()

    def prep(g, b):
        def cvt(k, c):
            row = g * L_BLK + lax.shift_right_logical(k, 3)
            col = lax.mul(lax.rem(k, 8), 16)
            SUP[b][pl.ds(k * 16, 16)] = idx_v[row, pl.ds(col, 16)]
            return c
        lax.fori_loop(0, NVREG, cvt, 0, unroll=2)

    def gather_start(b):
        pltpu.async_copy(tbl_hbm.at[SUP[b]], G[b], GSEM[b])

    def gather_wait(b):
        pltpu.make_async_copy(tbl_hbm.at[SUP[b]], G[b], GSEM[b]).wait()

    def scatter_start(g, b):
        pltpu.async_copy(O[b].at[:, :, :, pl.ds(0, BPW)],
                         out_hbm.at[pl.ds(g * L_BLK, L_BLK), :, wid],
                         SSEM[b])

    def scatter_wait(g, b):
        pltpu.make_async_copy(O[b].at[:, :, :, pl.ds(0, BPW)],
                              out_hbm.at[pl.ds(g * L_BLK, L_BLK), :, wid],
                              SSEM[b]).wait()

    iota = lax.iota(jnp.int32, 16)
    jt_lo = lax.shift_right_logical(iota, 3)
    js_lo = lax.rem(iota, 8)
    jt_hi = jt_lo + 2
    js_hi = js_lo

    def compute(g, b):
        gb, ob = G[b], O[b]
        for dl in range(L_BLK):
            l = g * L_BLK + dl
            p0 = pos_v[l, pl.ds(0, 16)]
            p1 = pos_v[l, pl.ds(16, 16)]
            dlv = jnp.full((16,), dl, jnp.int32)

            def bloop(bl, c):
                row = dl * BPW + bl
                blv = jnp.full((16,), bl, jnp.int32)
                plsc.store_scatter(ob, [dlv, jt_lo, js_lo, blv],
                                   gb[row, pl.ds(0, 16)] + p0)
                plsc.store_scatter(ob, [dlv, jt_hi, js_hi, blv],
                                   gb[row, pl.ds(16, 16)] + p1)
                return c

            lax.fori_loop(0, BPW, bloop, 0, unroll=4)

    for b in range(NBUF):
        prep(b, b)
        gather_start(b)

    def step(t, carry):
        for b in range(NBUF):
            g = t * NBUF + b
            gather_wait(b)

            @pl.when(t > 0)
            def _():
                scatter_wait(g - NBUF, b)

            compute(g, b)
            scatter_start(g, b)

            @pl.when(t < NUM_STEPS - 1)
            def _():
                prep(g + NBUF, b)
                gather_start(b)

        return carry

    lax.fori_loop(0, NUM_STEPS, step, 0, unroll=False)

    for b in range(NBUF):
        scatter_wait(NUM_CHUNKS - NBUF + b, b)


@jax.jit
def kernel(inputs, table, pos):
    idxT = inputs.T
    mesh = plsc.VectorSubcoreMesh(core_axis_name="c", subcore_axis_name="s")
    out6 = pl.kernel(
        _body,
        out_type=jax.ShapeDtypeStruct(
            (SEQ_LEN, EMBED_DIM // 8, NUM_WORKERS, 8, BPW), jnp.float32),
        mesh=mesh,
        compiler_params=pltpu.CompilerParams(use_tc_tiling_on_sc=False,
                                             needs_layout_passes=False),
        scratch_types=[
            pltpu.VMEM((SEQ_LEN, BPW), jnp.int32),
            pltpu.VMEM((SEQ_LEN, EMBED_DIM), jnp.float32),
        ] + [pltpu.VMEM((CHUNK, EMBED_DIM), jnp.float32)] * NBUF
          + [pltpu.VMEM((L_BLK, EMBED_DIM // 8, 8, OPITCH),
                        jnp.float32)] * NBUF
          + [pltpu.VMEM((CHUNK,), jnp.int32)] * NBUF
          + [pltpu.SemaphoreType.DMA] * (1 + 2 * NBUF),
    )(idxT, table, pos)
    return out6.transpose(2, 4, 0, 1, 3).reshape(BATCH, SEQ_LEN, EMBED_DIM)


# speedup vs baseline: 1.0069x; 1.0069x over previous
import jax
import jax.numpy as jnp
from jax import lax
from jax.experimental import pallas as pl
from jax.experimental.pallas import tpu as pltpu
from jax.experimental.pallas import tpu_sc as plsc

SEQ_LEN = 200
EMBED_DIM = 32
BATCH = 4096

NUM_CORES = 2
NUM_SUBCORES = 16
NUM_WORKERS = NUM_CORES * NUM_SUBCORES
BPW = BATCH // NUM_WORKERS

L_BLK = 2
CHUNK = L_BLK * BPW
NUM_CHUNKS = SEQ_LEN // L_BLK
NBUF = 2
NUM_STEPS = NUM_CHUNKS // NBUF
NVREG = CHUNK // 16
OPITCH = BPW + 1


def _body(idxT_hbm, tbl_hbm, pos_hbm, out_hbm,
          idx_v, pos_v, g0, g1, o0, o1, sup0, sup1,
          psem, gsem0, gsem1, ssem0, ssem1):
    G = (g0, g1)
    O = (o0, o1)
    SUP = (sup0, sup1)
    GSEM = (gsem0, gsem1)
    SSEM = (ssem0, ssem1)

    wid = lax.axis_index("s") * NUM_CORES + lax.axis_index("c")
    bbase = wid * BPW

    pltpu.sync_copy(pos_hbm, pos_v)
    pltpu.async_copy(idxT_hbm.at[:, pl.ds(bbase, BPW)], idx_v, psem).wait()

    def prep(g, b):
        def cvt(k, c):
            row = g * L_BLK + lax.shift_right_logical(k, 3)
            col = lax.mul(lax.rem(k, 8), 16)
            SUP[b][pl.ds(k * 16, 16)] = idx_v[row, pl.ds(col, 16)]
            return c
        lax.fori_loop(0, NVREG, cvt, 0, unroll=4)

    def gather_start(b):
        pltpu.async_copy(tbl_hbm.at[SUP[b]], G[b], GSEM[b])

    def gather_wait(b):
        pltpu.make_async_copy(tbl_hbm.at[SUP[b]], G[b], GSEM[b]).wait()

    def scatter_start(g, b):
        pltpu.async_copy(O[b].at[:, :, :, pl.ds(0, BPW)],
                         out_hbm.at[pl.ds(g * L_BLK, L_BLK), :, wid],
                         SSEM[b])

    def scatter_wait(g, b):
        pltpu.make_async_copy(O[b].at[:, :, :, pl.ds(0, BPW)],
                              out_hbm.at[pl.ds(g * L_BLK, L_BLK), :, wid],
                              SSEM[b]).wait()

    iota = lax.iota(jnp.int32, 16)
    jt_lo = lax.shift_right_logical(iota, 3)
    js_lo = lax.rem(iota, 8)
    jt_hi = jt_lo + 2
    js_hi = js_lo

    def compute(g, b):
        gb, ob = G[b], O[b]
        for dl in range(L_BLK):
            l = g * L_BLK + dl
            p0 = pos_v[l, pl.ds(0, 16)]
            p1 = pos_v[l, pl.ds(16, 16)]
            dlv = jnp.full((16,), dl, jnp.int32)

            def bloop(bl, c):
                row = dl * BPW + bl
                blv = jnp.full((16,), bl, jnp.int32)
                plsc.store_scatter(ob, [dlv, jt_lo, js_lo, blv],
                                   gb[row, pl.ds(0, 16)] + p0)
                plsc.store_scatter(ob, [dlv, jt_hi, js_hi, blv],
                                   gb[row, pl.ds(16, 16)] + p1)
                return c

            lax.fori_loop(0, BPW, bloop, 0, unroll=8)

    for b in range(NBUF):
        prep(b, b)
        gather_start(b)

    def step(t, carry):
        for b in range(NBUF):
            g = t * NBUF + b
            gather_wait(b)

            @pl.when(t > 0)
            def _():
                scatter_wait(g - NBUF, b)

            compute(g, b)
            scatter_start(g, b)

            @pl.when(t < NUM_STEPS - 1)
            def _():
                prep(g + NBUF, b)
                gather_start(b)

        return carry

    lax.fori_loop(0, NUM_STEPS, step, 0, unroll=False)

    for b in range(NBUF):
        scatter_wait(NUM_CHUNKS - NBUF + b, b)


@jax.jit
def kernel(inputs, table, pos):
    idxT = inputs.T
    mesh = plsc.VectorSubcoreMesh(core_axis_name="c", subcore_axis_name="s")
    out6 = pl.kernel(
        _body,
        out_type=jax.ShapeDtypeStruct(
            (SEQ_LEN, EMBED_DIM // 8, NUM_WORKERS, 8, BPW), jnp.float32),
        mesh=mesh,
        compiler_params=pltpu.CompilerParams(use_tc_tiling_on_sc=False,
                                             needs_layout_passes=False),
        scratch_types=[
            pltpu.VMEM((SEQ_LEN, BPW), jnp.int32),
            pltpu.VMEM((SEQ_LEN, EMBED_DIM), jnp.float32),
            pltpu.VMEM((CHUNK, EMBED_DIM), jnp.float32),
            pltpu.VMEM((CHUNK, EMBED_DIM), jnp.float32),
            pltpu.VMEM((L_BLK, EMBED_DIM // 8, 8, OPITCH), jnp.float32),
            pltpu.VMEM((L_BLK, EMBED_DIM // 8, 8, OPITCH), jnp.float32),
            pltpu.VMEM((CHUNK,), jnp.int32),
            pltpu.VMEM((CHUNK,), jnp.int32),
            pltpu.SemaphoreType.DMA,
            pltpu.SemaphoreType.DMA,
            pltpu.SemaphoreType.DMA,
            pltpu.SemaphoreType.DMA,
            pltpu.SemaphoreType.DMA,
        ],
    )(idxT, table, pos)
    return out6.transpose(2, 4, 0, 1, 3).reshape(BATCH, SEQ_LEN, EMBED_DIM)
